# baseline (speedup 1.0000x reference)
import functools

import jax
import jax.numpy as jnp
from jax import lax
from jax.experimental import pallas as pl
from jax.experimental.pallas import tpu as pltpu
from jax.experimental.pallas import tpu_sc as plsc

N_NODES = 10000
N_EDGES = 320000
D = 128
CHUNK = 112
NW = 32
CPT = 90
N_CHUNKS = NW * CPT
PAD = N_CHUNKS * CHUNK - N_EDGES
NBUF = 3
WB = 40
N_WB = N_NODES // WB


def _sc_aggregate(features, meta, wts):
    mesh = plsc.VectorSubcoreMesh(core_axis_name="c", subcore_axis_name="s")

    @functools.partial(
        pl.kernel,
        mesh=mesh,
        out_type=jax.ShapeDtypeStruct((2, N_NODES, D), jnp.float32),
        scratch_types=[
            pltpu.VMEM((NBUF, 2, CHUNK), jnp.int32),
            pltpu.VMEM((NBUF, 2, CHUNK), jnp.int32),
            pltpu.VMEM((NBUF, 1, CHUNK), jnp.float32),
            pltpu.VMEM((NBUF, 1, CHUNK), jnp.float32),
            pltpu.VMEM((CHUNK, D), jnp.float32),
            pltpu.VMEM((CHUNK, D), jnp.float32),
            pltpu.VMEM((CHUNK, D), jnp.float32),
            pltpu.VMEM((WB, D), jnp.float32),
            pltpu.VMEM((CHUNK,), jnp.int32),
            pltpu.VMEM((CHUNK,), jnp.int32),
            pltpu.VMEM((CHUNK,), jnp.int32),
            pltpu.VMEM_SHARED((N_NODES, D), jnp.float32),
            pltpu.SemaphoreType.DMA,
            pltpu.SemaphoreType.DMA,
            pltpu.SemaphoreType.DMA,
            pltpu.SemaphoreType.DMA,
            pltpu.SemaphoreType.DMA,
            pltpu.SemaphoreType.DMA,
            pltpu.SemaphoreType.DMA,
            pltpu.SemaphoreType.DMA,
        ],
    )
    def agg(feat_hbm, meta_hbm, wts_hbm, out_hbm,
            mb0, mb1, wb0, wb1, r0, r1, r2, bounce, d0, d1, d2, acc,
            m0, m1, g0, g1, g2, s0, s1, s2):
        mbuf = [mb0, mb1]
        wbuf = [wb0, wb1]
        rows = [r0, r1, r2]
        dstr = [d0, d1, d2]
        msem = [m0, m1]
        gsem = [g0, g1, g2]
        ssem = [s0, s1, s2]
        c = lax.axis_index("c")
        s = lax.axis_index("s")
        wid = s * 2 + c
        c0 = wid * CPT

        pltpu.async_copy(meta_hbm.at[pl.ds(c0, NBUF)], mb0, m0)
        pltpu.async_copy(wts_hbm.at[pl.ds(c0, NBUF)], wb0, m0)

        zero = jnp.zeros((16,), jnp.float32)

        def zrow(r, _):
            for j in range(D // 16):
                bounce[r, pl.ds(j * 16, 16)] = zero
            return 0

        lax.fori_loop(0, WB, zrow, 0)

        for k in range(N_WB // 16 + 1):
            blk = s + k * 16

            @pl.when(blk < N_WB)
            def _():
                pltpu.sync_copy(bounce, acc.at[pl.ds(blk * WB, WB)])

        pltpu.make_async_copy(meta_hbm.at[pl.ds(c0, NBUF)], mb0, m0).wait()
        pltpu.make_async_copy(wts_hbm.at[pl.ds(c0, NBUF)], wb0, m0).wait()
        pltpu.async_copy(feat_hbm.at[mb0.at[0, 0]], r0, g0)
        pltpu.async_copy(feat_hbm.at[mb0.at[1, 0]], r1, g1)
        plsc.subcore_barrier()

        def pair_body(t2, _):
            for u in range(2):
                m = u
                m2 = 1 - u
                for b in range(NBUF):
                    t = t2 * 2 + u
                    k = t * NBUF + b

                    if b == 0:
                        @pl.when(t < CPT // NBUF - 1)
                        def _():
                            pltpu.async_copy(
                                meta_hbm.at[pl.ds(c0 + (t + 1) * NBUF, NBUF)],
                                mbuf[m2], msem[m2])
                            pltpu.async_copy(
                                wts_hbm.at[pl.ds(c0 + (t + 1) * NBUF, NBUF)],
                                wbuf[m2], msem[m2])

                    bn = (b + 2) % NBUF

                    @pl.when(k + 2 < CPT)
                    def _():
                        @pl.when(k >= 1)
                        def _():
                            pltpu.make_async_copy(
                                rows[bn], acc.at[dstr[bn]], ssem[bn]).wait()

                        if b == 1:
                            pltpu.make_async_copy(
                                meta_hbm.at[pl.ds(c0 + (t + 1) * NBUF, NBUF)],
                                mbuf[m2], msem[m2]).wait()
                            pltpu.make_async_copy(
                                wts_hbm.at[pl.ds(c0 + (t + 1) * NBUF, NBUF)],
                                wbuf[m2], msem[m2]).wait()
                        src_ref = (mbuf[m].at[2, 0] if b == 0
                                   else mbuf[m2].at[b - 1, 0])
                        pltpu.async_copy(
                            feat_hbm.at[src_ref], rows[bn], gsem[bn])

                    pltpu.make_async_copy(
                        feat_hbm.at[mbuf[m].at[b, 0]], rows[b],
                        gsem[b]).wait()

                    @plsc.parallel_loop(0, CHUNK // 16, unroll=2)
                    def scale_16(k16):
                        wv = wbuf[m][b, 0, pl.ds(k16 * 16, 16)]
                        dstr[b][pl.ds(k16 * 16, 16)] = \
                            mbuf[m][b, 1, pl.ds(k16 * 16, 16)]
                        for i in range(16):
                            wi = wv[i]
                            e = k16 * 16 + i
                            for j in range(D // 16):
                                sl = pl.ds(j * 16, 16)
                                rows[b][e, sl] = rows[b][e, sl] * wi
                    pltpu.async_copy(
                        rows[b], acc.at[dstr[b]], ssem[b], add=True)
            return 0

        lax.fori_loop(0, CPT // (2 * NBUF), pair_body, 0)
        for b in range(NBUF):
            pltpu.make_async_copy(rows[b], acc.at[dstr[b]], ssem[b]).wait()
        plsc.subcore_barrier()

        for k in range(N_WB // 16 + 1):
            blk = s + k * 16

            @pl.when(blk < N_WB)
            def _():
                pltpu.sync_copy(acc.at[pl.ds(blk * WB, WB)], bounce)
                pltpu.sync_copy(bounce, out_hbm.at[c, pl.ds(blk * WB, WB)])

    return agg(features, meta, wts)


def _tc_linear_body(p_ref, f_ref, w_ref, b_ref, o_ref):
    x = p_ref[0] + p_ref[1] + f_ref[...]
    o_ref[...] = lax.dot_general(
        x, w_ref[...], (((1,), (1,)), ((), ())),
        preferred_element_type=jnp.float32) + b_ref[...]


def kernel(features, edge_index, edge_weight, W, b):
    src = jnp.pad(edge_index[0].astype(jnp.int32), (0, PAD))
    dst = jnp.pad(edge_index[1].astype(jnp.int32), (0, PAD))
    meta = jnp.stack(
        [src.reshape(N_CHUNKS, CHUNK),
         dst.reshape(N_CHUNKS, CHUNK)], axis=1)
    wts = jnp.pad(edge_weight.astype(jnp.float32),
                  (0, PAD)).reshape(N_CHUNKS, 1, CHUNK)

    partials = _sc_aggregate(features, meta, wts)

    blk = 2000
    out = pl.pallas_call(
        _tc_linear_body,
        grid=(N_NODES // blk,),
        in_specs=[
            pl.BlockSpec((2, blk, D), lambda i: (0, i, 0)),
            pl.BlockSpec((blk, D), lambda i: (i, 0)),
            pl.BlockSpec((D, D), lambda i: (0, 0)),
            pl.BlockSpec((1, D), lambda i: (0, 0)),
        ],
        out_specs=pl.BlockSpec((blk, D), lambda i: (i, 0)),
        out_shape=jax.ShapeDtypeStruct((N_NODES, D), jnp.float32),
    )(partials, features, W, b.reshape(1, D))
    return out


# speedup vs baseline: 1.0112x; 1.0112x over previous
import functools

import jax
import jax.numpy as jnp
from jax import lax
from jax.experimental import pallas as pl
from jax.experimental.pallas import tpu as pltpu
from jax.experimental.pallas import tpu_sc as plsc

N_NODES = 10000
N_EDGES = 320000
D = 128
CHUNK = 112
NW = 32
CPT = 90
N_CHUNKS = NW * CPT
PAD = N_CHUNKS * CHUNK - N_EDGES
NBUF = 3
WB = 40
N_WB = N_NODES // WB


def _sc_aggregate(features, meta, wts):
    mesh = plsc.VectorSubcoreMesh(core_axis_name="c", subcore_axis_name="s")

    @functools.partial(
        pl.kernel,
        mesh=mesh,
        out_type=jax.ShapeDtypeStruct((2, N_NODES, D), jnp.float32),
        scratch_types=[
            pltpu.VMEM((NBUF, 2, CHUNK), jnp.int32),
            pltpu.VMEM((NBUF, 2, CHUNK), jnp.int32),
            pltpu.VMEM((NBUF, 1, CHUNK), jnp.float32),
            pltpu.VMEM((NBUF, 1, CHUNK), jnp.float32),
            pltpu.VMEM((CHUNK, D), jnp.float32),
            pltpu.VMEM((CHUNK, D), jnp.float32),
            pltpu.VMEM((CHUNK, D), jnp.float32),
            pltpu.VMEM((WB, D), jnp.float32),
            pltpu.VMEM((CHUNK,), jnp.int32),
            pltpu.VMEM((CHUNK,), jnp.int32),
            pltpu.VMEM((CHUNK,), jnp.int32),
            pltpu.VMEM_SHARED((N_NODES, D), jnp.float32),
            pltpu.SemaphoreType.DMA,
            pltpu.SemaphoreType.DMA,
            pltpu.SemaphoreType.DMA,
            pltpu.SemaphoreType.DMA,
            pltpu.SemaphoreType.DMA,
            pltpu.SemaphoreType.DMA,
            pltpu.SemaphoreType.DMA,
            pltpu.SemaphoreType.DMA,
        ],
    )
    def agg(feat_hbm, meta_hbm, wts_hbm, out_hbm,
            mb0, mb1, wb0, wb1, r0, r1, r2, bounce, d0, d1, d2, acc,
            m0, m1, g0, g1, g2, s0, s1, s2):
        mbuf = [mb0, mb1]
        wbuf = [wb0, wb1]
        rows = [r0, r1, r2]
        dstr = [d0, d1, d2]
        msem = [m0, m1]
        gsem = [g0, g1, g2]
        ssem = [s0, s1, s2]
        c = lax.axis_index("c")
        s = lax.axis_index("s")
        wid = s * 2 + c
        c0 = wid * CPT

        pltpu.async_copy(meta_hbm.at[pl.ds(c0, NBUF)], mb0, m0)
        pltpu.async_copy(wts_hbm.at[pl.ds(c0, NBUF)], wb0, m0)

        zero = jnp.zeros((16,), jnp.float32)

        def zrow(r, _):
            for j in range(D // 16):
                bounce[r, pl.ds(j * 16, 16)] = zero
            return 0

        lax.fori_loop(0, WB, zrow, 0)

        for k in range(N_WB // 16 + 1):
            blk = s + k * 16

            @pl.when(blk < N_WB)
            def _():
                pltpu.sync_copy(bounce, acc.at[pl.ds(blk * WB, WB)])

        pltpu.make_async_copy(meta_hbm.at[pl.ds(c0, NBUF)], mb0, m0).wait()
        pltpu.make_async_copy(wts_hbm.at[pl.ds(c0, NBUF)], wb0, m0).wait()
        pltpu.async_copy(feat_hbm.at[mb0.at[0, 0]], r0, g0)
        pltpu.async_copy(feat_hbm.at[mb0.at[1, 0]], r1, g1)
        plsc.subcore_barrier()

        def pair_body(t2, _):
            for u in range(2):
                m = u
                m2 = 1 - u
                for b in range(NBUF):
                    t = t2 * 2 + u
                    k = t * NBUF + b

                    if b == 0:
                        @pl.when(t < CPT // NBUF - 1)
                        def _():
                            pltpu.async_copy(
                                meta_hbm.at[pl.ds(c0 + (t + 1) * NBUF, NBUF)],
                                mbuf[m2], msem[m2])
                            pltpu.async_copy(
                                wts_hbm.at[pl.ds(c0 + (t + 1) * NBUF, NBUF)],
                                wbuf[m2], msem[m2])

                    bn = (b + 2) % NBUF

                    pltpu.make_async_copy(
                        feat_hbm.at[mbuf[m].at[b, 0]], rows[b],
                        gsem[b]).wait()

                    @plsc.parallel_loop(0, CHUNK // 16, unroll=2)
                    def scale_16(k16):
                        wv = wbuf[m][b, 0, pl.ds(k16 * 16, 16)]
                        dstr[b][pl.ds(k16 * 16, 16)] = \
                            mbuf[m][b, 1, pl.ds(k16 * 16, 16)]
                        for i in range(16):
                            wi = wv[i]
                            e = k16 * 16 + i
                            for j in range(D // 16):
                                sl = pl.ds(j * 16, 16)
                                rows[b][e, sl] = rows[b][e, sl] * wi

                    @pl.when(k + 2 < CPT)
                    def _():
                        @pl.when(k >= 1)
                        def _():
                            pltpu.make_async_copy(
                                rows[bn], acc.at[dstr[bn]], ssem[bn]).wait()

                        if b == 1:
                            pltpu.make_async_copy(
                                meta_hbm.at[pl.ds(c0 + (t + 1) * NBUF, NBUF)],
                                mbuf[m2], msem[m2]).wait()
                            pltpu.make_async_copy(
                                wts_hbm.at[pl.ds(c0 + (t + 1) * NBUF, NBUF)],
                                wbuf[m2], msem[m2]).wait()
                        src_ref = (mbuf[m].at[2, 0] if b == 0
                                   else mbuf[m2].at[b - 1, 0])
                        pltpu.async_copy(
                            feat_hbm.at[src_ref], rows[bn], gsem[bn])

                    pltpu.async_copy(
                        rows[b], acc.at[dstr[b]], ssem[b], add=True)
            return 0

        lax.fori_loop(0, CPT // (2 * NBUF), pair_body, 0)
        for b in range(NBUF):
            pltpu.make_async_copy(rows[b], acc.at[dstr[b]], ssem[b]).wait()
        plsc.subcore_barrier()

        for k in range(N_WB // 16 + 1):
            blk = s + k * 16

            @pl.when(blk < N_WB)
            def _():
                pltpu.sync_copy(acc.at[pl.ds(blk * WB, WB)], bounce)
                pltpu.sync_copy(bounce, out_hbm.at[c, pl.ds(blk * WB, WB)])

    return agg(features, meta, wts)


def _tc_linear_body(p_ref, f_ref, w_ref, b_ref, o_ref):
    x = p_ref[0] + p_ref[1] + f_ref[...]
    o_ref[...] = lax.dot_general(
        x, w_ref[...], (((1,), (1,)), ((), ())),
        preferred_element_type=jnp.float32) + b_ref[...]


def kernel(features, edge_index, edge_weight, W, b):
    src = jnp.pad(edge_index[0].astype(jnp.int32), (0, PAD))
    dst = jnp.pad(edge_index[1].astype(jnp.int32), (0, PAD))
    meta = jnp.stack(
        [src.reshape(N_CHUNKS, CHUNK),
         dst.reshape(N_CHUNKS, CHUNK)], axis=1)
    wts = jnp.pad(edge_weight.astype(jnp.float32),
                  (0, PAD)).reshape(N_CHUNKS, 1, CHUNK)

    partials = _sc_aggregate(features, meta, wts)

    blk = 2000
    out = pl.pallas_call(
        _tc_linear_body,
        grid=(N_NODES // blk,),
        in_specs=[
            pl.BlockSpec((2, blk, D), lambda i: (0, i, 0)),
            pl.BlockSpec((blk, D), lambda i: (i, 0)),
            pl.BlockSpec((D, D), lambda i: (0, 0)),
            pl.BlockSpec((1, D), lambda i: (0, 0)),
        ],
        out_specs=pl.BlockSpec((blk, D), lambda i: (i, 0)),
        out_shape=jax.ShapeDtypeStruct((N_NODES, D), jnp.float32),
    )(partials, features, W, b.reshape(1, D))
    return out
